# baseline (speedup 1.0000x reference)
import jax
import jax.numpy as jnp
from jax import lax
from jax.experimental import pallas as pl
from jax.experimental.pallas import tpu as pltpu
from jax.experimental.pallas import tpu_sc as plsc

_LANES = 16
_UNROLL = 16
_CHUNKS = (16, 16, 18)


def _gather_chunk(E_T, idx_T, h0, hc, V, D, B, idx_dtype):
    mesh = plsc.VectorSubcoreMesh(core_axis_name="core",
                                  subcore_axis_name="subcore")
    n_sub = 32
    d_per = D // n_sub

    @pl.kernel(
        out_type=jax.ShapeDtypeStruct((hc, D, B), E_T.dtype),
        mesh=mesh,
        scratch_types=[
            pltpu.VMEM((V,), E_T.dtype),
            pltpu.VMEM((B,), idx_dtype),
            pltpu.VMEM((B,), idx_dtype),
            pltpu.VMEM((B,), E_T.dtype),
            pltpu.VMEM((B,), E_T.dtype),
            pltpu.SemaphoreType.DMA,
            pltpu.SemaphoreType.DMA,
            pltpu.SemaphoreType.DMA,
            pltpu.SemaphoreType.DMA,
            pltpu.SemaphoreType.DMA,
        ],
        compiler_params=pltpu.CompilerParams(use_tc_tiling_on_sc=False,
                                             needs_layout_passes=False),
    )
    def gather_kernel(et_hbm, it_hbm, o_hbm, row, ib0, ib1, ob0, ob1,
                      sem_row, sem_i0, sem_i1, sem_o0, sem_o1):
        c = lax.axis_index("core")
        s = lax.axis_index("subcore")
        t = c * 16 + s

        def gather_into(ob, ib):
            @plsc.parallel_loop(0, B, step=_LANES, unroll=_UNROLL)
            def _(i):
                sl = pl.ds(i, _LANES)
                ob[sl] = plsc.load_gather(row, [ib[sl]])

        @pl.loop(0, d_per)
        def _(j):
            d = t * d_per + j
            pltpu.make_async_copy(et_hbm.at[d], row, sem_row).start()
            pltpu.make_async_copy(it_hbm.at[h0], ib0, sem_i0).start()
            pltpu.make_async_copy(it_hbm.at[h0 + 1], ib1, sem_i1).start()
            pltpu.make_async_copy(et_hbm.at[d], row, sem_row).wait()

            @pl.loop(0, hc // 2)
            def _(hh):
                g0 = 2 * hh
                g1 = g0 + 1

                pltpu.make_async_copy(it_hbm.at[h0 + g0], ib0, sem_i0).wait()

                @pl.when(hh > 0)
                def _():
                    pltpu.make_async_copy(ob0, o_hbm.at[g0 - 2, d],
                                          sem_o0).wait()

                gather_into(ob0, ib0)
                pltpu.make_async_copy(ob0, o_hbm.at[g0, d], sem_o0).start()

                @pl.when(g0 + 2 < hc)
                def _():
                    pltpu.make_async_copy(it_hbm.at[h0 + g0 + 2], ib0,
                                          sem_i0).start()

                pltpu.make_async_copy(it_hbm.at[h0 + g1], ib1, sem_i1).wait()

                @pl.when(hh > 0)
                def _():
                    pltpu.make_async_copy(ob1, o_hbm.at[g1 - 2, d],
                                          sem_o1).wait()

                gather_into(ob1, ib1)
                pltpu.make_async_copy(ob1, o_hbm.at[g1, d], sem_o1).start()

                @pl.when(g1 + 2 < hc)
                def _():
                    pltpu.make_async_copy(it_hbm.at[h0 + g1 + 2], ib1,
                                          sem_i1).start()

            pltpu.make_async_copy(ob0, o_hbm.at[hc - 2, d], sem_o0).wait()
            pltpu.make_async_copy(ob1, o_hbm.at[hc - 1, d], sem_o1).wait()

    return gather_kernel(E_T, idx_T)


def kernel(indices, E):
    B, H = indices.shape
    V, D = E.shape
    assert sum(_CHUNKS) == H
    E_T = E.T
    idx_T = indices.T

    parts = []
    h0 = 0
    for hc in _CHUNKS:
        out_c = _gather_chunk(E_T, idx_T, h0, hc, V, D, B, indices.dtype)
        parts.append(jnp.transpose(out_c, (2, 0, 1)))
        h0 += hc
    return jnp.concatenate(parts, axis=1)


# speedup vs baseline: 1.2714x; 1.2714x over previous
import jax
import jax.numpy as jnp
from jax import lax
from jax.experimental import pallas as pl
from jax.experimental.pallas import tpu as pltpu
from jax.experimental.pallas import tpu_sc as plsc

_LANES = 16
_UNROLL = 16


def kernel(indices, E):
    B, H = indices.shape
    V, D = E.shape
    E_T = E.T
    idx_T = indices.T

    mesh = plsc.VectorSubcoreMesh(core_axis_name="core",
                                  subcore_axis_name="subcore")
    n_sub = 32
    d_per = D // n_sub

    @pl.kernel(
        out_type=jax.ShapeDtypeStruct((H, D, B), E.dtype),
        mesh=mesh,
        scratch_types=[
            pltpu.VMEM((V,), E.dtype),
            pltpu.VMEM((B,), indices.dtype),
            pltpu.VMEM((B,), indices.dtype),
            pltpu.VMEM((B,), E.dtype),
            pltpu.VMEM((B,), E.dtype),
            pltpu.SemaphoreType.DMA,
            pltpu.SemaphoreType.DMA,
            pltpu.SemaphoreType.DMA,
            pltpu.SemaphoreType.DMA,
            pltpu.SemaphoreType.DMA,
        ],
        compiler_params=pltpu.CompilerParams(use_tc_tiling_on_sc=False,
                                             needs_layout_passes=False),
    )
    def gather_kernel(et_hbm, it_hbm, o_hbm, row, ib0, ib1, ob0, ob1,
                      sem_row, sem_i0, sem_i1, sem_o0, sem_o1):
        c = lax.axis_index("core")
        s = lax.axis_index("subcore")
        t = c * 16 + s

        def gather_into(ob, ib):
            @plsc.parallel_loop(0, B, step=_LANES, unroll=_UNROLL)
            def _(i):
                sl = pl.ds(i, _LANES)
                ob[sl] = plsc.load_gather(row, [ib[sl]])

        @pl.loop(0, d_per)
        def _(j):
            d = t * d_per + j
            pltpu.make_async_copy(et_hbm.at[d], row, sem_row).start()
            pltpu.make_async_copy(it_hbm.at[0], ib0, sem_i0).start()
            pltpu.make_async_copy(it_hbm.at[1], ib1, sem_i1).start()
            pltpu.make_async_copy(et_hbm.at[d], row, sem_row).wait()

            @pl.loop(0, H // 2)
            def _(hh):
                h0 = 2 * hh
                h1 = h0 + 1

                pltpu.make_async_copy(it_hbm.at[h0], ib0, sem_i0).wait()

                @pl.when(hh > 0)
                def _():
                    pltpu.make_async_copy(ob0, o_hbm.at[h0 - 2, d],
                                          sem_o0).wait()

                gather_into(ob0, ib0)
                pltpu.make_async_copy(ob0, o_hbm.at[h0, d], sem_o0).start()

                @pl.when(h0 + 2 < H)
                def _():
                    pltpu.make_async_copy(it_hbm.at[h0 + 2], ib0,
                                          sem_i0).start()

                pltpu.make_async_copy(it_hbm.at[h1], ib1, sem_i1).wait()

                @pl.when(hh > 0)
                def _():
                    pltpu.make_async_copy(ob1, o_hbm.at[h1 - 2, d],
                                          sem_o1).wait()

                gather_into(ob1, ib1)
                pltpu.make_async_copy(ob1, o_hbm.at[h1, d], sem_o1).start()

                @pl.when(h1 + 2 < H)
                def _():
                    pltpu.make_async_copy(it_hbm.at[h1 + 2], ib1,
                                          sem_i1).start()

            pltpu.make_async_copy(ob0, o_hbm.at[H - 2, d], sem_o0).wait()
            pltpu.make_async_copy(ob1, o_hbm.at[H - 1, d], sem_o1).wait()

    out = gather_kernel(E_T, idx_T)
    return jnp.transpose(out, (2, 0, 1))
